# Initial kernel scaffold; baseline (speedup 1.0000x reference)
import jax
import jax.numpy as jnp
from jax.experimental import pallas as pl


def kernel(ref_feat, e_weight, W, e_ref, e_query, e_kernel):
    raise NotImplementedError("write your pallas kernel here")


# speedup vs baseline: 1.6156x; 1.6156x over previous
import functools

import jax
import jax.numpy as jnp
from jax import lax
from jax.experimental import pallas as pl
from jax.experimental.pallas import tpu as pltpu
from jax.experimental.pallas import tpu_sc as plsc

_CHUNK = 128
_LANES = 16
_NC = 2
_NS = 16
_NW = _NC * _NS


_BN = 400


def _mm_body(x_ref, w_ref, h_ref):
    h_ref[0] = jnp.dot(x_ref[...], w_ref[0], preferred_element_type=jnp.float32)


def _make_h(ref_feat, W, interpret=False):
    N, CIN = ref_feat.shape
    K, _, D = W.shape
    return pl.pallas_call(
        _mm_body,
        grid=(N // _BN, K),
        in_specs=[
            pl.BlockSpec((_BN, CIN), lambda i, k: (i, 0)),
            pl.BlockSpec((1, CIN, D), lambda i, k: (k, 0, 0)),
        ],
        out_specs=pl.BlockSpec((1, _BN, D), lambda i, k: (k, i, 0)),
        out_shape=jax.ShapeDtypeStruct((K, N, D), jnp.float32),
        interpret=interpret,
    )(ref_feat, W)


def _sc_edge_call(h_flat, idx, q, w, zeros_rows, N):
    NK, D = h_flat.shape
    EP = idx.shape[0]
    ncht = EP // _CHUNK
    rpt = zeros_rows.shape[0]
    rlast = N - rpt * (_NS - 1)

    mesh = plsc.VectorSubcoreMesh(core_axis_name="c", subcore_axis_name="s")

    @functools.partial(
        pl.kernel,
        out_type=jax.ShapeDtypeStruct((_NC, N, D), jnp.float32),
        mesh=mesh,
        scratch_types=[
            pltpu.VMEM((_CHUNK,), jnp.int32),
            pltpu.VMEM((_CHUNK,), jnp.int32),
            pltpu.VMEM((_CHUNK,), jnp.float32),
            pltpu.VMEM((_CHUNK, D), jnp.float32),
            pltpu.VMEM_SHARED((N, D), jnp.float32),
            pltpu.SemaphoreType.DMA,
        ],
    )
    def body(h_hbm, idx_hbm, q_hbm, w_hbm, z_hbm, out_hbm,
             idx_v, q_v, w_v, rows_v, acc_sh, sem):
        c = lax.axis_index("c")
        s = lax.axis_index("s")
        wid = c * _NS + s

        @pl.when(s < _NS - 1)
        def _zero_full():
            pltpu.sync_copy(z_hbm, acc_sh.at[pl.ds(s * rpt, rpt)])

        @pl.when(s == _NS - 1)
        def _zero_last():
            pltpu.sync_copy(z_hbm.at[pl.ds(0, rlast)],
                            acc_sh.at[pl.ds((_NS - 1) * rpt, rlast)])

        plsc.subcore_barrier()

        nloc = (ncht - wid + _NW - 1) // _NW

        def chunk(t, carry):
            base = (wid + t * _NW) * _CHUNK
            pltpu.sync_copy(idx_hbm.at[pl.ds(base, _CHUNK)], idx_v)
            pltpu.sync_copy(w_hbm.at[pl.ds(base, _CHUNK)], w_v)
            pltpu.sync_copy(q_hbm.at[pl.ds(base, _CHUNK)], q_v)
            pltpu.async_copy(h_hbm.at[idx_v], rows_v, sem).wait()

            def grp(j, cc):
                wv = w_v[pl.ds(j * _LANES, _LANES)]
                for l in range(_LANES):
                    wl = lax.gather(
                        wv, jnp.full((_LANES, 1), l, jnp.int32),
                        lax.GatherDimensionNumbers(
                            offset_dims=(), collapsed_slice_dims=(0,),
                            start_index_map=(0,)),
                        slice_sizes=(1,),
                        mode=lax.GatherScatterMode.PROMISE_IN_BOUNDS)
                    e = j * _LANES + l
                    for sg in range(D // _LANES):
                        sl = pl.ds(sg * _LANES, _LANES)
                        rows_v[e, sl] = rows_v[e, sl] * wl
                return cc

            lax.fori_loop(0, _CHUNK // _LANES, grp, 0)

            pltpu.sync_copy(rows_v, acc_sh.at[q_v], add=True)
            return carry

        lax.fori_loop(0, nloc, chunk, 0)
        plsc.subcore_barrier()

        @pl.when(s < _NS - 1)
        def _out_full():
            pltpu.sync_copy(acc_sh.at[pl.ds(s * rpt, rpt)],
                            out_hbm.at[c, pl.ds(s * rpt, rpt)])

        @pl.when(s == _NS - 1)
        def _out_last():
            pltpu.sync_copy(acc_sh.at[pl.ds((_NS - 1) * rpt, rlast)],
                            out_hbm.at[c, pl.ds((_NS - 1) * rpt, rlast)])

    return body(h_flat, idx, q, w, zeros_rows)


_BN2 = 2000


def _combine(parts, interpret=False):
    _, N, D = parts.shape

    def body(p_ref, o_ref):
        o_ref[...] = jnp.maximum(p_ref[0] + p_ref[1], 0.0)

    return pl.pallas_call(
        body,
        grid=(N // _BN2,),
        in_specs=[pl.BlockSpec((_NC, _BN2, D), lambda i: (0, i, 0))],
        out_specs=pl.BlockSpec((_BN2, D), lambda i: (i, 0)),
        out_shape=jax.ShapeDtypeStruct((N, D), jnp.float32),
        interpret=interpret,
    )(parts)


def kernel(ref_feat, e_weight, W, e_ref, e_query, e_kernel):
    N, _ = ref_feat.shape
    K, _, D = W.shape
    E = e_ref.shape[0]

    h = _make_h(ref_feat, W)
    h_flat = h.reshape(K * N, D)

    idx = e_kernel.astype(jnp.int32) * N + e_ref.astype(jnp.int32)
    q = e_query.astype(jnp.int32)
    w = e_weight.astype(jnp.float32)

    ep = -(-E // _CHUNK) * _CHUNK
    if ep != E:
        pad = ep - E
        idx = jnp.concatenate([idx, jnp.arange(pad, dtype=jnp.int32) % (K * N)])
        q = jnp.concatenate([q, jnp.arange(pad, dtype=jnp.int32) % N])
        w = jnp.concatenate([w, jnp.zeros((pad,), jnp.float32)])

    rpt = (-(-N // _NS) + 7) // 8 * 8
    zeros_rows = jnp.zeros((rpt, D), jnp.float32)
    parts = _sc_edge_call(h_flat, idx, q, w, zeros_rows, N)
    return _combine(parts)


# speedup vs baseline: 2.6334x; 1.6300x over previous
import functools

import jax
import jax.numpy as jnp
from jax import lax
from jax.experimental import pallas as pl
from jax.experimental.pallas import tpu as pltpu
from jax.experimental.pallas import tpu_sc as plsc

_CHUNK = 128
_LANES = 16
_NC = 2
_NS = 16
_NW = _NC * _NS


_BN = 2000


def _mm_body(x_ref, w_ref, h_ref):
    h_ref[0] = jnp.dot(x_ref[...], w_ref[0], preferred_element_type=jnp.float32)


def _make_h(ref_feat, W, interpret=False):
    N, CIN = ref_feat.shape
    K, _, D = W.shape
    return pl.pallas_call(
        _mm_body,
        grid=(N // _BN, K),
        in_specs=[
            pl.BlockSpec((_BN, CIN), lambda i, k: (i, 0)),
            pl.BlockSpec((1, CIN, D), lambda i, k: (k, 0, 0)),
        ],
        out_specs=pl.BlockSpec((1, _BN, D), lambda i, k: (k, i, 0)),
        out_shape=jax.ShapeDtypeStruct((K, N, D), jnp.float32),
        interpret=interpret,
    )(ref_feat, W)


def _sc_edge_call(h_flat, idx, q, w, zeros_rows, N):
    NK, D = h_flat.shape
    EP = idx.shape[0]
    ncht = EP // _CHUNK
    rpt = zeros_rows.shape[0]
    rlast = N - rpt * (_NS - 1)

    mesh = plsc.VectorSubcoreMesh(core_axis_name="c", subcore_axis_name="s")

    @functools.partial(
        pl.kernel,
        out_type=jax.ShapeDtypeStruct((_NC, N, D), jnp.float32),
        mesh=mesh,
        scratch_types=[
            pltpu.VMEM((_CHUNK,), jnp.int32),
            pltpu.VMEM((_CHUNK,), jnp.int32),
            pltpu.VMEM((_CHUNK,), jnp.float32),
            pltpu.VMEM((_CHUNK, D), jnp.float32),
            pltpu.VMEM_SHARED((N, D), jnp.float32),
            pltpu.SemaphoreType.DMA,
        ],
    )
    def body(h_hbm, idx_hbm, q_hbm, w_hbm, z_hbm, out_hbm,
             idx_v, q_v, w_v, rows_v, acc_sh, sem):
        c = lax.axis_index("c")
        s = lax.axis_index("s")
        wid = c * _NS + s

        @pl.when(s < _NS - 1)
        def _zero_full():
            pltpu.sync_copy(z_hbm, acc_sh.at[pl.ds(s * rpt, rpt)])

        @pl.when(s == _NS - 1)
        def _zero_last():
            pltpu.sync_copy(z_hbm.at[pl.ds(0, rlast)],
                            acc_sh.at[pl.ds((_NS - 1) * rpt, rlast)])

        plsc.subcore_barrier()

        nloc = (ncht - wid + _NW - 1) // _NW

        def chunk(t, carry):
            base = (wid + t * _NW) * _CHUNK
            pltpu.sync_copy(idx_hbm.at[pl.ds(base, _CHUNK)], idx_v)
            pltpu.sync_copy(w_hbm.at[pl.ds(base, _CHUNK)], w_v)
            pltpu.sync_copy(q_hbm.at[pl.ds(base, _CHUNK)], q_v)
            pltpu.async_copy(h_hbm.at[idx_v], rows_v, sem).wait()

            def grp(j, cc):
                wv = w_v[pl.ds(j * _LANES, _LANES)]
                for l in range(_LANES):
                    wl = lax.gather(
                        wv, jnp.full((_LANES, 1), l, jnp.int32),
                        lax.GatherDimensionNumbers(
                            offset_dims=(), collapsed_slice_dims=(0,),
                            start_index_map=(0,)),
                        slice_sizes=(1,),
                        mode=lax.GatherScatterMode.PROMISE_IN_BOUNDS)
                    e = j * _LANES + l
                    for sg in range(D // _LANES):
                        sl = pl.ds(sg * _LANES, _LANES)
                        rows_v[e, sl] = rows_v[e, sl] * wl
                return cc

            lax.fori_loop(0, _CHUNK // _LANES, grp, 0)

            pltpu.sync_copy(rows_v, acc_sh.at[q_v], add=True)
            return carry

        lax.fori_loop(0, nloc, chunk, 0)
        plsc.subcore_barrier()

        @pl.when(s < _NS - 1)
        def _out_full():
            pltpu.sync_copy(acc_sh.at[pl.ds(s * rpt, rpt)],
                            out_hbm.at[c, pl.ds(s * rpt, rpt)])

        @pl.when(s == _NS - 1)
        def _out_last():
            pltpu.sync_copy(acc_sh.at[pl.ds((_NS - 1) * rpt, rlast)],
                            out_hbm.at[c, pl.ds((_NS - 1) * rpt, rlast)])

    return body(h_flat, idx, q, w, zeros_rows)


_BN2 = 2000


def _combine(parts, interpret=False):
    _, N, D = parts.shape

    def body(p_ref, o_ref):
        o_ref[...] = jnp.maximum(p_ref[0] + p_ref[1], 0.0)

    return pl.pallas_call(
        body,
        grid=(N // _BN2,),
        in_specs=[pl.BlockSpec((_NC, _BN2, D), lambda i: (0, i, 0))],
        out_specs=pl.BlockSpec((_BN2, D), lambda i: (i, 0)),
        out_shape=jax.ShapeDtypeStruct((N, D), jnp.float32),
        interpret=interpret,
    )(parts)


def kernel(ref_feat, e_weight, W, e_ref, e_query, e_kernel):
    N, _ = ref_feat.shape
    K, _, D = W.shape
    E = e_ref.shape[0]

    h = _make_h(ref_feat, W)
    h_flat = h.reshape(K * N, D)

    idx = e_kernel.astype(jnp.int32) * N + e_ref.astype(jnp.int32)
    q = e_query.astype(jnp.int32)
    w = e_weight.astype(jnp.float32)

    ep = -(-E // _CHUNK) * _CHUNK
    if ep != E:
        pad = ep - E
        idx = jnp.concatenate([idx, jnp.arange(pad, dtype=jnp.int32) % (K * N)])
        q = jnp.concatenate([q, jnp.arange(pad, dtype=jnp.int32) % N])
        w = jnp.concatenate([w, jnp.zeros((pad,), jnp.float32)])

    rpt = (-(-N // _NS) + 7) // 8 * 8
    zeros_rows = jnp.zeros((rpt, D), jnp.float32)
    parts = _sc_edge_call(h_flat, idx, q, w, zeros_rows, N)
    return _combine(parts)


# speedup vs baseline: 4.0537x; 1.5393x over previous
import functools

import jax
import jax.numpy as jnp
from jax import lax
from jax.experimental import pallas as pl
from jax.experimental.pallas import tpu as pltpu
from jax.experimental.pallas import tpu_sc as plsc

_CHUNK = 128
_LANES = 16
_NC = 2
_NS = 16
_NW = _NC * _NS


_BN = 2000


def _mm_body(x_ref, w_ref, h_ref):
    h_ref[0] = jnp.dot(x_ref[...], w_ref[0], preferred_element_type=jnp.float32)


def _make_h(ref_feat, W, interpret=False):
    N, CIN = ref_feat.shape
    K, _, D = W.shape
    return pl.pallas_call(
        _mm_body,
        grid=(N // _BN, K),
        in_specs=[
            pl.BlockSpec((_BN, CIN), lambda i, k: (i, 0)),
            pl.BlockSpec((1, CIN, D), lambda i, k: (k, 0, 0)),
        ],
        out_specs=pl.BlockSpec((1, _BN, D), lambda i, k: (k, i, 0)),
        out_shape=jax.ShapeDtypeStruct((K, N, D), jnp.float32),
        interpret=interpret,
    )(ref_feat, W)


def _scale_rows(rows_b, wb):
    def grp(j, cc):
        wv = wb[pl.ds(j * _LANES, _LANES)]
        for l in range(_LANES):
            wl = lax.gather(
                wv, jnp.full((_LANES, 1), l, jnp.int32),
                lax.GatherDimensionNumbers(
                    offset_dims=(), collapsed_slice_dims=(0,),
                    start_index_map=(0,)),
                slice_sizes=(1,),
                mode=lax.GatherScatterMode.PROMISE_IN_BOUNDS)
            e = j * _LANES + l
            for sg in range(_D_OUT // _LANES):
                sl = pl.ds(sg * _LANES, _LANES)
                rows_b[e, sl] = rows_b[e, sl] * wl
        return cc

    lax.fori_loop(0, _CHUNK // _LANES, grp, 0)


_D_OUT = 128


def _sc_edge_call(h_flat, meta, wch, zeros_rows, N):
    NK, D = h_flat.shape
    ncht = meta.shape[0]
    rpt = zeros_rows.shape[0]
    rlast = N - rpt * (_NS - 1)
    ntw = -(-ncht // _NW)
    nt = ((ntw + 1) // 2) * 2

    mesh = plsc.VectorSubcoreMesh(core_axis_name="c", subcore_axis_name="s")

    @functools.partial(
        pl.kernel,
        out_type=jax.ShapeDtypeStruct((_NC, N, D), jnp.float32),
        mesh=mesh,
        scratch_types=[
            pltpu.VMEM((2, _CHUNK), jnp.int32),
            pltpu.VMEM((2, _CHUNK), jnp.int32),
            pltpu.VMEM((_CHUNK,), jnp.float32),
            pltpu.VMEM((_CHUNK,), jnp.float32),
            pltpu.VMEM((_CHUNK, D), jnp.float32),
            pltpu.VMEM((_CHUNK, D), jnp.float32),
            pltpu.VMEM_SHARED((N, D), jnp.float32),
            pltpu.SemaphoreType.DMA,
            pltpu.SemaphoreType.DMA,
            pltpu.SemaphoreType.DMA,
            pltpu.SemaphoreType.DMA,
            pltpu.SemaphoreType.DMA,
            pltpu.SemaphoreType.DMA,
        ],
    )
    def body(h_hbm, meta_hbm, w_hbm, z_hbm, out_hbm,
             mbA, mbB, wbA, wbB, rowsA, rowsB, acc_sh,
             semA, semB, wsemA, wsemB, gsemA, gsemB):
        c = lax.axis_index("c")
        s = lax.axis_index("s")
        wid = c * _NS + s
        mb = (mbA, mbB)
        wb = (wbA, wbB)
        rows = (rowsA, rowsB)
        msem = (semA, semB)
        wsem = (wsemA, wsemB)
        gsem = (gsemA, gsemB)

        @pl.when(s < _NS - 1)
        def _zero_full():
            pltpu.sync_copy(z_hbm, acc_sh.at[pl.ds(s * rpt, rpt)])

        @pl.when(s == _NS - 1)
        def _zero_last():
            pltpu.sync_copy(z_hbm.at[pl.ds(0, rlast)],
                            acc_sh.at[pl.ds((_NS - 1) * rpt, rlast)])

        plsc.subcore_barrier()

        def cid(t):
            return wid + t * _NW

        def valid(t):
            return cid(t) < ncht

        pltpu.sync_copy(meta_hbm.at[cid(0)], mbA)
        pltpu.sync_copy(w_hbm.at[cid(0)], wbA)
        pltpu.async_copy(h_hbm.at[mbA.at[0]], rowsA, gsemA)

        @pl.when(valid(1))
        def _meta1():
            pltpu.async_copy(meta_hbm.at[cid(1)], mbB, semB)
            pltpu.async_copy(w_hbm.at[cid(1)], wbB, wsemB)

        def step(t, b):
            nb = 1 - b

            @pl.when(valid(t + 1))
            def _fire_next():
                pltpu.make_async_copy(meta_hbm.at[cid(t + 1)], mb[nb],
                                      msem[nb]).wait()
                pltpu.make_async_copy(w_hbm.at[cid(t + 1)], wb[nb],
                                      wsem[nb]).wait()
                pltpu.async_copy(h_hbm.at[mb[nb].at[0]], rows[nb], gsem[nb])

            @pl.when(valid(t))
            def _process():
                pltpu.make_async_copy(h_hbm.at[mb[b].at[0]], rows[b],
                                      gsem[b]).wait()
                _scale_rows(rows[b], wb[b])
                pltpu.sync_copy(rows[b], acc_sh.at[mb[b].at[1]], add=True)

            @pl.when(valid(t + 2))
            def _meta_next2():
                pltpu.async_copy(meta_hbm.at[cid(t + 2)], mb[b], msem[b])
                pltpu.async_copy(w_hbm.at[cid(t + 2)], wb[b], wsem[b])

        def pair(t2, carry):
            step(t2 * 2, 0)
            step(t2 * 2 + 1, 1)
            return carry

        lax.fori_loop(0, nt // 2, pair, 0)
        plsc.subcore_barrier()

        @pl.when(s < _NS - 1)
        def _out_full():
            pltpu.sync_copy(acc_sh.at[pl.ds(s * rpt, rpt)],
                            out_hbm.at[c, pl.ds(s * rpt, rpt)])

        @pl.when(s == _NS - 1)
        def _out_last():
            pltpu.sync_copy(acc_sh.at[pl.ds((_NS - 1) * rpt, rlast)],
                            out_hbm.at[c, pl.ds((_NS - 1) * rpt, rlast)])

    return body(h_flat, meta, wch, zeros_rows)


_BN2 = 2000


def _combine(parts, interpret=False):
    _, N, D = parts.shape

    def body(p_ref, o_ref):
        o_ref[...] = jnp.maximum(p_ref[0] + p_ref[1], 0.0)

    return pl.pallas_call(
        body,
        grid=(N // _BN2,),
        in_specs=[pl.BlockSpec((_NC, _BN2, D), lambda i: (0, i, 0))],
        out_specs=pl.BlockSpec((_BN2, D), lambda i: (i, 0)),
        out_shape=jax.ShapeDtypeStruct((N, D), jnp.float32),
        interpret=interpret,
    )(parts)


def kernel(ref_feat, e_weight, W, e_ref, e_query, e_kernel):
    N, _ = ref_feat.shape
    K, _, D = W.shape
    E = e_ref.shape[0]

    h = _make_h(ref_feat, W)
    h_flat = h.reshape(K * N, D)

    idx = e_kernel.astype(jnp.int32) * N + e_ref.astype(jnp.int32)
    q = e_query.astype(jnp.int32)
    w = e_weight.astype(jnp.float32)

    ep = -(-E // _CHUNK) * _CHUNK
    if ep != E:
        pad = ep - E
        idx = jnp.concatenate([idx, jnp.arange(pad, dtype=jnp.int32) % (K * N)])
        q = jnp.concatenate([q, jnp.arange(pad, dtype=jnp.int32) % N])
        w = jnp.concatenate([w, jnp.zeros((pad,), jnp.float32)])

    ncht = ep // _CHUNK
    meta = jnp.stack([
        idx.reshape(ncht, _CHUNK),
        q.reshape(ncht, _CHUNK),
    ], axis=1)
    wch = w.reshape(ncht, _CHUNK)

    rpt = (-(-N // _NS) + 7) // 8 * 8
    zeros_rows = jnp.zeros((rpt, D), jnp.float32)
    parts = _sc_edge_call(h_flat, meta, wch, zeros_rows, N)
    return _combine(parts)


# speedup vs baseline: 4.4889x; 1.1074x over previous
import functools

import jax
import jax.numpy as jnp
from jax import lax
from jax.experimental import pallas as pl
from jax.experimental.pallas import tpu as pltpu
from jax.experimental.pallas import tpu_sc as plsc

_CHUNK = 128
_LANES = 16
_NC = 2
_NS = 16
_NW = _NC * _NS


_BN = 2000


def _mm_body(x_ref, w_ref, h_ref):
    h_ref[0] = jnp.dot(x_ref[...], w_ref[0], preferred_element_type=jnp.float32)


def _make_h(ref_feat, W, interpret=False):
    N, CIN = ref_feat.shape
    K, _, D = W.shape
    return pl.pallas_call(
        _mm_body,
        grid=(N // _BN, K),
        in_specs=[
            pl.BlockSpec((_BN, CIN), lambda i, k: (i, 0)),
            pl.BlockSpec((1, CIN, D), lambda i, k: (k, 0, 0)),
        ],
        out_specs=pl.BlockSpec((1, _BN, D), lambda i, k: (k, i, 0)),
        out_shape=jax.ShapeDtypeStruct((K, N, D), jnp.float32),
        interpret=interpret,
    )(ref_feat, W)


def _scale_rows(rows_b, wb):
    def grp(j, cc):
        wv = wb[pl.ds(j * _LANES, _LANES)]
        for l in range(_LANES):
            wl = lax.gather(
                wv, jnp.full((_LANES, 1), l, jnp.int32),
                lax.GatherDimensionNumbers(
                    offset_dims=(), collapsed_slice_dims=(0,),
                    start_index_map=(0,)),
                slice_sizes=(1,),
                mode=lax.GatherScatterMode.PROMISE_IN_BOUNDS)
            e = j * _LANES + l
            for sg in range(_D_OUT // _LANES):
                sl = pl.ds(sg * _LANES, _LANES)
                rows_b[e, sl] = rows_b[e, sl] * wl
        return cc

    lax.fori_loop(0, _CHUNK // _LANES, grp, 0)


_D_OUT = 128


def _sc_edge_call(h_flat, meta, wch, zeros_rows, N):
    NK, D = h_flat.shape
    ncht = meta.shape[0]
    rpt = zeros_rows.shape[0]
    rlast = N - rpt * (_NS - 1)
    ntw = -(-ncht // _NW)
    nt = ((ntw + 1) // 2) * 2

    mesh = plsc.VectorSubcoreMesh(core_axis_name="c", subcore_axis_name="s")

    @functools.partial(
        pl.kernel,
        out_type=jax.ShapeDtypeStruct((_NC, N, D), jnp.float32),
        mesh=mesh,
        scratch_types=[
            pltpu.VMEM((2, _CHUNK), jnp.int32),
            pltpu.VMEM((2, _CHUNK), jnp.int32),
            pltpu.VMEM((_CHUNK,), jnp.float32),
            pltpu.VMEM((_CHUNK,), jnp.float32),
            pltpu.VMEM((_CHUNK, D), jnp.float32),
            pltpu.VMEM((_CHUNK, D), jnp.float32),
            pltpu.VMEM((_CHUNK,), jnp.int32),
            pltpu.VMEM((_CHUNK,), jnp.int32),
            pltpu.VMEM_SHARED((N, D), jnp.float32),
            pltpu.SemaphoreType.DMA,
            pltpu.SemaphoreType.DMA,
            pltpu.SemaphoreType.DMA,
            pltpu.SemaphoreType.DMA,
            pltpu.SemaphoreType.DMA,
            pltpu.SemaphoreType.DMA,
            pltpu.SemaphoreType.DMA,
            pltpu.SemaphoreType.DMA,
        ],
    )
    def body(h_hbm, meta_hbm, w_hbm, z_hbm, out_hbm,
             mbA, mbB, wbA, wbB, rowsA, rowsB, qbA, qbB, acc_sh,
             semA, semB, wsemA, wsemB, gsemA, gsemB, ssemA, ssemB):
        c = lax.axis_index("c")
        s = lax.axis_index("s")
        wid = c * _NS + s
        mb = (mbA, mbB)
        wb = (wbA, wbB)
        rows = (rowsA, rowsB)
        qb = (qbA, qbB)
        msem = (semA, semB)
        wsem = (wsemA, wsemB)
        gsem = (gsemA, gsemB)
        ssem = (ssemA, ssemB)

        @pl.when(s < _NS - 1)
        def _zero_full():
            pltpu.sync_copy(z_hbm, acc_sh.at[pl.ds(s * rpt, rpt)])

        @pl.when(s == _NS - 1)
        def _zero_last():
            pltpu.sync_copy(z_hbm.at[pl.ds(0, rlast)],
                            acc_sh.at[pl.ds((_NS - 1) * rpt, rlast)])

        plsc.subcore_barrier()

        def cid(t):
            return wid + t * _NW

        def valid(t):
            return cid(t) < ncht

        pltpu.sync_copy(meta_hbm.at[cid(0)], mbA)
        pltpu.sync_copy(w_hbm.at[cid(0)], wbA)
        pltpu.async_copy(h_hbm.at[mbA.at[0]], rowsA, gsemA)

        @pl.when(valid(1))
        def _meta1():
            pltpu.async_copy(meta_hbm.at[cid(1)], mbB, semB)
            pltpu.async_copy(w_hbm.at[cid(1)], wbB, wsemB)

        def step(t2, t, b):
            nb = 1 - b

            @pl.when(valid(t + 1))
            def _fire_next():
                pltpu.make_async_copy(meta_hbm.at[cid(t + 1)], mb[nb],
                                      msem[nb]).wait()
                pltpu.make_async_copy(w_hbm.at[cid(t + 1)], wb[nb],
                                      wsem[nb]).wait()
                if b == 0:
                    @pl.when(t2 > 0)
                    def _drain_prev():
                        pltpu.make_async_copy(
                            rows[nb], acc_sh.at[qb[nb]], ssem[nb]).wait()
                else:
                    pltpu.make_async_copy(
                        rows[nb], acc_sh.at[qb[nb]], ssem[nb]).wait()
                pltpu.async_copy(h_hbm.at[mb[nb].at[0]], rows[nb], gsem[nb])

            @pl.when(valid(t))
            def _process():
                pltpu.make_async_copy(h_hbm.at[mb[b].at[0]], rows[b],
                                      gsem[b]).wait()
                _scale_rows(rows[b], wb[b])
                for k in range(_CHUNK // _LANES):
                    sl = pl.ds(k * _LANES, _LANES)
                    qb[b][sl] = mb[b][1, sl]
                pltpu.async_copy(rows[b], acc_sh.at[qb[b]], ssem[b], add=True)

            @pl.when(valid(t + 2))
            def _meta_next2():
                pltpu.async_copy(meta_hbm.at[cid(t + 2)], mb[b], msem[b])
                pltpu.async_copy(w_hbm.at[cid(t + 2)], wb[b], wsem[b])

        def pair(t2, carry):
            step(t2, t2 * 2, 0)
            step(t2, t2 * 2 + 1, 1)
            return carry

        lax.fori_loop(0, nt // 2, pair, 0)
        for b in (0, 1):
            pltpu.make_async_copy(rows[b], acc_sh.at[qb[b]], ssem[b]).wait()
        plsc.subcore_barrier()

        @pl.when(s < _NS - 1)
        def _out_full():
            pltpu.sync_copy(acc_sh.at[pl.ds(s * rpt, rpt)],
                            out_hbm.at[c, pl.ds(s * rpt, rpt)])

        @pl.when(s == _NS - 1)
        def _out_last():
            pltpu.sync_copy(acc_sh.at[pl.ds((_NS - 1) * rpt, rlast)],
                            out_hbm.at[c, pl.ds((_NS - 1) * rpt, rlast)])

    return body(h_flat, meta, wch, zeros_rows)


_BN2 = 2000


def _combine(parts, interpret=False):
    _, N, D = parts.shape

    def body(p_ref, o_ref):
        o_ref[...] = jnp.maximum(p_ref[0] + p_ref[1], 0.0)

    return pl.pallas_call(
        body,
        grid=(N // _BN2,),
        in_specs=[pl.BlockSpec((_NC, _BN2, D), lambda i: (0, i, 0))],
        out_specs=pl.BlockSpec((_BN2, D), lambda i: (i, 0)),
        out_shape=jax.ShapeDtypeStruct((N, D), jnp.float32),
        interpret=interpret,
    )(parts)


def kernel(ref_feat, e_weight, W, e_ref, e_query, e_kernel):
    N, _ = ref_feat.shape
    K, _, D = W.shape
    E = e_ref.shape[0]

    h = _make_h(ref_feat, W)
    h_flat = h.reshape(K * N, D)

    idx = e_kernel.astype(jnp.int32) * N + e_ref.astype(jnp.int32)
    q = e_query.astype(jnp.int32)
    w = e_weight.astype(jnp.float32)

    ep = -(-E // _CHUNK) * _CHUNK
    if ep != E:
        pad = ep - E
        idx = jnp.concatenate([idx, jnp.arange(pad, dtype=jnp.int32) % (K * N)])
        q = jnp.concatenate([q, jnp.arange(pad, dtype=jnp.int32) % N])
        w = jnp.concatenate([w, jnp.zeros((pad,), jnp.float32)])

    ncht = ep // _CHUNK
    meta = jnp.stack([
        idx.reshape(ncht, _CHUNK),
        q.reshape(ncht, _CHUNK),
    ], axis=1)
    wch = w.reshape(ncht, _CHUNK)

    rpt = (-(-N // _NS) + 7) // 8 * 8
    zeros_rows = jnp.zeros((rpt, D), jnp.float32)
    parts = _sc_edge_call(h_flat, meta, wch, zeros_rows, N)
    return _combine(parts)


# speedup vs baseline: 5.7181x; 1.2738x over previous
import functools

import jax
import jax.numpy as jnp
from jax import lax
from jax.experimental import pallas as pl
from jax.experimental.pallas import tpu as pltpu
from jax.experimental.pallas import tpu_sc as plsc

_CHUNK = 128
_LANES = 16
_NC = 2
_NS = 16
_NW = _NC * _NS


_BN = 1000


def _mm_body(x_ref, w_ref, h_ref):
    for k in range(w_ref.shape[0]):
        h_ref[k] = jnp.dot(x_ref[...], w_ref[k],
                           preferred_element_type=jnp.float32)


def _make_h(ref_feat, W, interpret=False):
    N, CIN = ref_feat.shape
    K, _, D = W.shape
    return pl.pallas_call(
        _mm_body,
        grid=(N // _BN,),
        in_specs=[
            pl.BlockSpec((_BN, CIN), lambda i: (i, 0)),
            pl.BlockSpec((K, CIN, D), lambda i: (0, 0, 0)),
        ],
        out_specs=pl.BlockSpec((K, _BN, D), lambda i: (0, i, 0)),
        out_shape=jax.ShapeDtypeStruct((K, N, D), jnp.float32),
        interpret=interpret,
    )(ref_feat, W)


def _scale_rows(rows_b, wb):
    def grp(j, cc):
        wv = wb[pl.ds(j * _LANES, _LANES)]
        for l in range(_LANES):
            wl = lax.gather(
                wv, jnp.full((_LANES, 1), l, jnp.int32),
                lax.GatherDimensionNumbers(
                    offset_dims=(), collapsed_slice_dims=(0,),
                    start_index_map=(0,)),
                slice_sizes=(1,),
                mode=lax.GatherScatterMode.PROMISE_IN_BOUNDS)
            e = j * _LANES + l
            for sg in range(_D_OUT // _LANES):
                sl = pl.ds(sg * _LANES, _LANES)
                rows_b[e, sl] = rows_b[e, sl] * wl
        return cc

    lax.fori_loop(0, _CHUNK // _LANES, grp, 0)


_D_OUT = 128


def _sc_edge_call(h_flat, meta, wch, zeros_rows, N):
    NK, D = h_flat.shape
    ncht = meta.shape[0]
    rpt = zeros_rows.shape[0]
    rlast = N - rpt * (_NS - 1)
    ntw = -(-ncht // _NW)
    nt = ((ntw + 1) // 2) * 2

    mesh = plsc.VectorSubcoreMesh(core_axis_name="c", subcore_axis_name="s")

    @functools.partial(
        pl.kernel,
        out_type=jax.ShapeDtypeStruct((_NC, N, D), jnp.float32),
        mesh=mesh,
        scratch_types=[
            pltpu.VMEM((2, _CHUNK), jnp.int32),
            pltpu.VMEM((2, _CHUNK), jnp.int32),
            pltpu.VMEM((_CHUNK,), jnp.float32),
            pltpu.VMEM((_CHUNK,), jnp.float32),
            pltpu.VMEM((_CHUNK, D), jnp.float32),
            pltpu.VMEM((_CHUNK, D), jnp.float32),
            pltpu.VMEM((_CHUNK,), jnp.int32),
            pltpu.VMEM((_CHUNK,), jnp.int32),
            pltpu.VMEM_SHARED((N, D), jnp.float32),
            pltpu.SemaphoreType.DMA,
            pltpu.SemaphoreType.DMA,
            pltpu.SemaphoreType.DMA,
            pltpu.SemaphoreType.DMA,
            pltpu.SemaphoreType.DMA,
            pltpu.SemaphoreType.DMA,
            pltpu.SemaphoreType.DMA,
            pltpu.SemaphoreType.DMA,
        ],
    )
    def body(h_hbm, meta_hbm, w_hbm, z_hbm, out_hbm,
             mbA, mbB, wbA, wbB, rowsA, rowsB, qbA, qbB, acc_sh,
             semA, semB, wsemA, wsemB, gsemA, gsemB, ssemA, ssemB):
        c = lax.axis_index("c")
        s = lax.axis_index("s")
        wid = c * _NS + s
        mb = (mbA, mbB)
        wb = (wbA, wbB)
        rows = (rowsA, rowsB)
        qb = (qbA, qbB)
        msem = (semA, semB)
        wsem = (wsemA, wsemB)
        gsem = (gsemA, gsemB)
        ssem = (ssemA, ssemB)

        @pl.when(s < _NS - 1)
        def _zero_full():
            pltpu.sync_copy(z_hbm, acc_sh.at[pl.ds(s * rpt, rpt)])

        @pl.when(s == _NS - 1)
        def _zero_last():
            pltpu.sync_copy(z_hbm.at[pl.ds(0, rlast)],
                            acc_sh.at[pl.ds((_NS - 1) * rpt, rlast)])

        plsc.subcore_barrier()

        def cid(t):
            return wid + t * _NW

        def valid(t):
            return cid(t) < ncht

        pltpu.sync_copy(meta_hbm.at[cid(0)], mbA)
        pltpu.sync_copy(w_hbm.at[cid(0)], wbA)
        pltpu.async_copy(h_hbm.at[mbA.at[0]], rowsA, gsemA)

        @pl.when(valid(1))
        def _meta1():
            pltpu.async_copy(meta_hbm.at[cid(1)], mbB, semB)
            pltpu.async_copy(w_hbm.at[cid(1)], wbB, wsemB)

        def step(t2, t, b):
            nb = 1 - b

            @pl.when(valid(t + 1))
            def _fire_next():
                pltpu.make_async_copy(meta_hbm.at[cid(t + 1)], mb[nb],
                                      msem[nb]).wait()
                pltpu.make_async_copy(w_hbm.at[cid(t + 1)], wb[nb],
                                      wsem[nb]).wait()
                if b == 0:
                    @pl.when(t2 > 0)
                    def _drain_prev():
                        pltpu.make_async_copy(
                            rows[nb], acc_sh.at[qb[nb]], ssem[nb]).wait()
                else:
                    pltpu.make_async_copy(
                        rows[nb], acc_sh.at[qb[nb]], ssem[nb]).wait()
                pltpu.async_copy(h_hbm.at[mb[nb].at[0]], rows[nb], gsem[nb])

            @pl.when(valid(t))
            def _process():
                pltpu.make_async_copy(h_hbm.at[mb[b].at[0]], rows[b],
                                      gsem[b]).wait()
                _scale_rows(rows[b], wb[b])
                for k in range(_CHUNK // _LANES):
                    sl = pl.ds(k * _LANES, _LANES)
                    qb[b][sl] = mb[b][1, sl]
                pltpu.async_copy(rows[b], acc_sh.at[qb[b]], ssem[b], add=True)

            @pl.when(valid(t + 2))
            def _meta_next2():
                pltpu.async_copy(meta_hbm.at[cid(t + 2)], mb[b], msem[b])
                pltpu.async_copy(w_hbm.at[cid(t + 2)], wb[b], wsem[b])

        def pair(t2, carry):
            step(t2, t2 * 2, 0)
            step(t2, t2 * 2 + 1, 1)
            return carry

        lax.fori_loop(0, nt // 2, pair, 0)
        for b in (0, 1):
            pltpu.make_async_copy(rows[b], acc_sh.at[qb[b]], ssem[b]).wait()
        plsc.subcore_barrier()

        @pl.when(s < _NS - 1)
        def _out_full():
            pltpu.sync_copy(acc_sh.at[pl.ds(s * rpt, rpt)],
                            out_hbm.at[c, pl.ds(s * rpt, rpt)])

        @pl.when(s == _NS - 1)
        def _out_last():
            pltpu.sync_copy(acc_sh.at[pl.ds((_NS - 1) * rpt, rlast)],
                            out_hbm.at[c, pl.ds((_NS - 1) * rpt, rlast)])

    return body(h_flat, meta, wch, zeros_rows)


_BN2 = 2000


def _combine(parts, interpret=False):
    _, N, D = parts.shape

    def body(p_ref, o_ref):
        o_ref[...] = jnp.maximum(p_ref[0] + p_ref[1], 0.0)

    return pl.pallas_call(
        body,
        grid=(N // _BN2,),
        in_specs=[pl.BlockSpec((_NC, _BN2, D), lambda i: (0, i, 0))],
        out_specs=pl.BlockSpec((_BN2, D), lambda i: (i, 0)),
        out_shape=jax.ShapeDtypeStruct((N, D), jnp.float32),
        interpret=interpret,
    )(parts)


def kernel(ref_feat, e_weight, W, e_ref, e_query, e_kernel):
    N, _ = ref_feat.shape
    K, _, D = W.shape
    E = e_ref.shape[0]

    h = _make_h(ref_feat, W)
    h_flat = h.reshape(K * N, D)

    idx = e_kernel.astype(jnp.int32) * N + e_ref.astype(jnp.int32)
    q = e_query.astype(jnp.int32)
    w = e_weight.astype(jnp.float32)

    ep = -(-E // _CHUNK) * _CHUNK
    if ep != E:
        pad = ep - E
        idx = jnp.concatenate([idx, jnp.arange(pad, dtype=jnp.int32) % (K * N)])
        q = jnp.concatenate([q, jnp.arange(pad, dtype=jnp.int32) % N])
        w = jnp.concatenate([w, jnp.zeros((pad,), jnp.float32)])

    ncht = ep // _CHUNK
    meta = jnp.stack([
        idx.reshape(ncht, _CHUNK),
        q.reshape(ncht, _CHUNK),
    ], axis=1)
    wch = w.reshape(ncht, _CHUNK)

    rpt = (-(-N // _NS) + 7) // 8 * 8
    zeros_rows = jnp.zeros((rpt, D), jnp.float32)
    parts = _sc_edge_call(h_flat, meta, wch, zeros_rows, N)
    return _combine(parts)


# speedup vs baseline: 6.0637x; 1.0604x over previous
import functools

import jax
import jax.numpy as jnp
from jax import lax
from jax.experimental import pallas as pl
from jax.experimental.pallas import tpu as pltpu
from jax.experimental.pallas import tpu_sc as plsc

_CHUNK = 128
_LANES = 16
_NC = 2
_NS = 16
_NW = _NC * _NS
_RB = 3


_BN = 1000


def _mm_body(x_ref, w_ref, h_ref):
    for k in range(w_ref.shape[0]):
        h_ref[k] = jnp.dot(x_ref[...], w_ref[k],
                           preferred_element_type=jnp.float32)


def _make_h(ref_feat, W, interpret=False):
    N, CIN = ref_feat.shape
    K, _, D = W.shape
    return pl.pallas_call(
        _mm_body,
        grid=(N // _BN,),
        in_specs=[
            pl.BlockSpec((_BN, CIN), lambda i: (i, 0)),
            pl.BlockSpec((K, CIN, D), lambda i: (0, 0, 0)),
        ],
        out_specs=pl.BlockSpec((K, _BN, D), lambda i: (0, i, 0)),
        out_shape=jax.ShapeDtypeStruct((K, N, D), jnp.float32),
        interpret=interpret,
    )(ref_feat, W)


def _scale_rows(rows_b, wb):
    def grp(j, cc):
        wv = wb[pl.ds(j * _LANES, _LANES)]
        for l in range(_LANES):
            wl = lax.gather(
                wv, jnp.full((_LANES, 1), l, jnp.int32),
                lax.GatherDimensionNumbers(
                    offset_dims=(), collapsed_slice_dims=(0,),
                    start_index_map=(0,)),
                slice_sizes=(1,),
                mode=lax.GatherScatterMode.PROMISE_IN_BOUNDS)
            e = j * _LANES + l
            for sg in range(_D_OUT // _LANES):
                sl = pl.ds(sg * _LANES, _LANES)
                rows_b[e, sl] = rows_b[e, sl] * wl
        return cc

    lax.fori_loop(0, _CHUNK // _LANES, grp, 0)


_D_OUT = 128


def _sc_edge_call(h_flat, meta, wch, zeros_rows, N):
    NK, D = h_flat.shape
    ncht = meta.shape[0]
    rpt = zeros_rows.shape[0]
    rlast = N - rpt * (_NS - 1)
    ntw = -(-ncht // _NW)
    nt = ((ntw + _RB - 1) // _RB) * _RB

    mesh = plsc.VectorSubcoreMesh(core_axis_name="c", subcore_axis_name="s")

    @functools.partial(
        pl.kernel,
        out_type=jax.ShapeDtypeStruct((_NC, N, D), jnp.float32),
        mesh=mesh,
        scratch_types=[
            [pltpu.VMEM((2, _CHUNK), jnp.int32) for _ in range(_RB)],
            [pltpu.VMEM((_CHUNK,), jnp.float32) for _ in range(_RB)],
            [pltpu.VMEM((_CHUNK, D), jnp.float32) for _ in range(_RB)],
            [pltpu.VMEM((_CHUNK,), jnp.int32) for _ in range(_RB)],
            pltpu.VMEM_SHARED((N, D), jnp.float32),
            [pltpu.SemaphoreType.DMA for _ in range(_RB)],
            [pltpu.SemaphoreType.DMA for _ in range(_RB)],
            [pltpu.SemaphoreType.DMA for _ in range(_RB)],
            [pltpu.SemaphoreType.DMA for _ in range(_RB)],
        ],
    )
    def body(h_hbm, meta_hbm, w_hbm, z_hbm, out_hbm,
             mb, wb, rows, qb, acc_sh, msem, wsem, gsem, ssem):
        c = lax.axis_index("c")
        s = lax.axis_index("s")
        wid = c * _NS + s

        @pl.when(s < _NS - 1)
        def _zero_full():
            pltpu.sync_copy(z_hbm, acc_sh.at[pl.ds(s * rpt, rpt)])

        @pl.when(s == _NS - 1)
        def _zero_last():
            pltpu.sync_copy(z_hbm.at[pl.ds(0, rlast)],
                            acc_sh.at[pl.ds((_NS - 1) * rpt, rlast)])

        plsc.subcore_barrier()

        def cid(t):
            return wid + t * _NW

        def valid(t):
            return cid(t) < ncht

        pltpu.sync_copy(meta_hbm.at[cid(0)], mb[0])
        pltpu.sync_copy(w_hbm.at[cid(0)], wb[0])
        pltpu.async_copy(h_hbm.at[mb[0].at[0]], rows[0], gsem[0])
        for r in range(1, _RB):
            @pl.when(valid(r))
            def _meta_pre(r=r):
                pltpu.async_copy(meta_hbm.at[cid(r)], mb[r], msem[r])
                pltpu.async_copy(w_hbm.at[cid(r)], wb[r], wsem[r])

        def step(t4, t, b):
            nb = (b + 1) % _RB

            @pl.when(valid(t + 1))
            def _fire_next():
                pltpu.make_async_copy(meta_hbm.at[cid(t + 1)], mb[nb],
                                      msem[nb]).wait()
                pltpu.make_async_copy(w_hbm.at[cid(t + 1)], wb[nb],
                                      wsem[nb]).wait()
                if b == _RB - 1:
                    pltpu.make_async_copy(
                        rows[nb], acc_sh.at[qb[nb]], ssem[nb]).wait()
                else:
                    @pl.when(t4 > 0)
                    def _drain_prev():
                        pltpu.make_async_copy(
                            rows[nb], acc_sh.at[qb[nb]], ssem[nb]).wait()
                pltpu.async_copy(h_hbm.at[mb[nb].at[0]], rows[nb], gsem[nb])

            @pl.when(valid(t))
            def _process():
                pltpu.make_async_copy(h_hbm.at[mb[b].at[0]], rows[b],
                                      gsem[b]).wait()
                _scale_rows(rows[b], wb[b])
                for k in range(_CHUNK // _LANES):
                    sl = pl.ds(k * _LANES, _LANES)
                    qb[b][sl] = mb[b][1, sl]
                pltpu.async_copy(rows[b], acc_sh.at[qb[b]], ssem[b], add=True)

            @pl.when(valid(t + _RB))
            def _meta_next():
                pltpu.async_copy(meta_hbm.at[cid(t + _RB)], mb[b], msem[b])
                pltpu.async_copy(w_hbm.at[cid(t + _RB)], wb[b], wsem[b])

        def ring(t4, carry):
            for b in range(_RB):
                step(t4, t4 * _RB + b, b)
            return carry

        lax.fori_loop(0, nt // _RB, ring, 0)
        for b in range(_RB):
            pltpu.make_async_copy(rows[b], acc_sh.at[qb[b]], ssem[b]).wait()
        plsc.subcore_barrier()

        @pl.when(s < _NS - 1)
        def _out_full():
            pltpu.sync_copy(acc_sh.at[pl.ds(s * rpt, rpt)],
                            out_hbm.at[c, pl.ds(s * rpt, rpt)])

        @pl.when(s == _NS - 1)
        def _out_last():
            pltpu.sync_copy(acc_sh.at[pl.ds((_NS - 1) * rpt, rlast)],
                            out_hbm.at[c, pl.ds((_NS - 1) * rpt, rlast)])

    return body(h_flat, meta, wch, zeros_rows)


_BN2 = 2000


def _combine(parts, interpret=False):
    _, N, D = parts.shape

    def body(p_ref, o_ref):
        o_ref[...] = jnp.maximum(p_ref[0] + p_ref[1], 0.0)

    return pl.pallas_call(
        body,
        grid=(N // _BN2,),
        in_specs=[pl.BlockSpec((_NC, _BN2, D), lambda i: (0, i, 0))],
        out_specs=pl.BlockSpec((_BN2, D), lambda i: (i, 0)),
        out_shape=jax.ShapeDtypeStruct((N, D), jnp.float32),
        interpret=interpret,
    )(parts)


def kernel(ref_feat, e_weight, W, e_ref, e_query, e_kernel):
    N, _ = ref_feat.shape
    K, _, D = W.shape
    E = e_ref.shape[0]

    h = _make_h(ref_feat, W)
    h_flat = h.reshape(K * N, D)

    idx = e_kernel.astype(jnp.int32) * N + e_ref.astype(jnp.int32)
    q = e_query.astype(jnp.int32)
    w = e_weight.astype(jnp.float32)

    ep = -(-E // _CHUNK) * _CHUNK
    if ep != E:
        pad = ep - E
        idx = jnp.concatenate([idx, jnp.arange(pad, dtype=jnp.int32) % (K * N)])
        q = jnp.concatenate([q, jnp.arange(pad, dtype=jnp.int32) % N])
        w = jnp.concatenate([w, jnp.zeros((pad,), jnp.float32)])

    ncht = ep // _CHUNK
    meta = jnp.stack([
        idx.reshape(ncht, _CHUNK),
        q.reshape(ncht, _CHUNK),
    ], axis=1)
    wch = w.reshape(ncht, _CHUNK)

    rpt = (-(-N // _NS) + 7) // 8 * 8
    zeros_rows = jnp.zeros((rpt, D), jnp.float32)
    parts = _sc_edge_call(h_flat, meta, wch, zeros_rows, N)
    return _combine(parts)
